# baseline (speedup 1.0000x reference)
import functools

import jax
import jax.numpy as jnp
from jax import lax
from jax.experimental import pallas as pl
from jax.experimental.pallas import tpu as pltpu
from jax.experimental.pallas import tpu_sc as plsc

H = 128
NC, NS = 2, 16
NW = NC * NS
CG = 32
CS = 64
BLK = 512

_F32 = jnp.float32


def _rup(x, m):
    return (x + m - 1) // m * m


def _pad_rows(x, p):
    return jnp.pad(x, ((0, p - x.shape[0]), (0, 0)))


def _pad_idx(x, p, val):
    return jnp.pad(x, (0, p - x.shape[0]), constant_values=val)


def _silu(x):
    return x * (1.0 / (1.0 + jnp.exp(-x)))


def _ln(x, g, b):
    mu = jnp.mean(x, axis=-1, keepdims=True)
    var = jnp.mean((x - mu) ** 2, axis=-1, keepdims=True)
    return (x - mu) / jnp.sqrt(var + 1e-5) * g + b


def _dot(a, b):
    return jnp.dot(a, b, preferred_element_type=_F32)


def _wspec():
    return pl.BlockSpec((H, H), lambda i: (0, 0))


def _vspec():
    return pl.BlockSpec((1, H), lambda i: (0, 0))


def _rspec(blk=BLK):
    return pl.BlockSpec((blk, H), lambda i: (i, 0))


def _mm_body(x_ref, w_ref, o_ref):
    o_ref[...] = _dot(x_ref[...], w_ref[...])


def _mm(x, w):
    n = x.shape[0]
    return pl.pallas_call(
        _mm_body,
        grid=(n // BLK,),
        in_specs=[_rspec(), _wspec()],
        out_specs=_rspec(),
        out_shape=jax.ShapeDtypeStruct((n, H), _F32),
    )(x, w)


def _tables_body(x_ref, ws_ref, wr_ref, xs_ref, xr_ref):
    x = x_ref[...]
    xs_ref[...] = _dot(x, ws_ref[...])
    xr_ref[...] = _dot(x, wr_ref[...])


def _tables(x, ws, wr):
    n = x.shape[0]
    return pl.pallas_call(
        _tables_body,
        grid=(n // BLK,),
        in_specs=[_rspec(), _wspec(), _wspec()],
        out_specs=[_rspec(), _rspec()],
        out_shape=[jax.ShapeDtypeStruct((n, H), _F32)] * 2,
    )(x, ws, wr)


def _edge_core(e_ref, g_ref, w1_ref, b1_ref, w2_ref, b2_ref, lg_ref, lb_ref):
    h = _dot(e_ref[...], w1_ref[...]) + g_ref[...] + b1_ref[...]
    h = _silu(h)
    m = _dot(h, w2_ref[...]) + b2_ref[...]
    return _ln(m, lg_ref[...], lb_ref[...])


def _edge_body(e_ref, g_ref, w1_ref, b1_ref, w2_ref, b2_ref, lg_ref, lb_ref,
               msg_ref):
    msg_ref[...] = _edge_core(e_ref, g_ref, w1_ref, b1_ref, w2_ref, b2_ref,
                              lg_ref, lb_ref)


def _edge_upd_body(e_ref, g_ref, w1_ref, b1_ref, w2_ref, b2_ref, lg_ref,
                   lb_ref, msg_ref, ne_ref):
    m = _edge_core(e_ref, g_ref, w1_ref, b1_ref, w2_ref, b2_ref, lg_ref,
                   lb_ref)
    msg_ref[...] = m
    ne_ref[...] = e_ref[...] + m


def _edge_mlp(edge, g, w1e, b1, w2, b2, lg, lb, update_edges):
    n = edge.shape[0]
    specs = [_rspec(), _rspec(), _wspec(), _vspec(), _wspec(), _vspec(),
             _vspec(), _vspec()]
    if update_edges:
        return pl.pallas_call(
            _edge_upd_body,
            grid=(n // BLK,),
            in_specs=specs,
            out_specs=[_rspec(), _rspec()],
            out_shape=[jax.ShapeDtypeStruct((n, H), _F32)] * 2,
        )(edge, g, w1e, b1, w2, b2, lg, lb)
    return pl.pallas_call(
        _edge_body,
        grid=(n // BLK,),
        in_specs=specs,
        out_specs=_rspec(),
        out_shape=jax.ShapeDtypeStruct((n, H), _F32),
    )(edge, g, w1e, b1, w2, b2, lg, lb)


def _node_core(rec, agg, wr_ref, wa_ref, b1_ref, w2_ref, b2_ref, lg_ref,
               lb_ref):
    h = _silu(_dot(rec, wr_ref[...]) + _dot(agg, wa_ref[...]) + b1_ref[...])
    m = _dot(h, w2_ref[...]) + b2_ref[...]
    return rec + _ln(m, lg_ref[...], lb_ref[...])


def _node2_body(rec_ref, parts_ref, wr_ref, wa_ref, b1_ref, w2_ref, b2_ref,
                lg_ref, lb_ref, o_ref):
    agg = parts_ref[0] + parts_ref[1]
    o_ref[...] = _node_core(rec_ref[...], agg, wr_ref, wa_ref, b1_ref, w2_ref,
                            b2_ref, lg_ref, lb_ref)


def _node2(rec, parts, wr, wa, b1, w2, b2, lg, lb):
    n = rec.shape[0]
    return pl.pallas_call(
        _node2_body,
        grid=(n // BLK,),
        in_specs=[_rspec(), pl.BlockSpec((2, BLK, H), lambda i: (0, i, 0)),
                  _wspec(), _wspec(), _vspec(), _wspec(), _vspec(), _vspec(),
                  _vspec()],
        out_specs=_rspec(),
        out_shape=jax.ShapeDtypeStruct((n, H), _F32),
    )(rec, parts, wr, wa, b1, w2, b2, lg, lb)


def _node1_body(rec_ref, agg_ref, wr_ref, wa_ref, b1_ref, w2_ref, b2_ref,
                lg_ref, lb_ref, o_ref):
    o_ref[...] = _node_core(rec_ref[...], agg_ref[...], wr_ref, wa_ref,
                            b1_ref, w2_ref, b2_ref, lg_ref, lb_ref)


def _node1(rec, agg, wr, wa, b1, w2, b2, lg, lb):
    n = rec.shape[0]
    return pl.pallas_call(
        _node1_body,
        grid=(n // BLK,),
        in_specs=[_rspec(), _rspec(), _wspec(), _wspec(), _vspec(), _wspec(),
                  _vspec(), _vspec(), _vspec()],
        out_specs=_rspec(),
        out_shape=jax.ShapeDtypeStruct((n, H), _F32),
    )(rec, agg, wr, wa, b1, w2, b2, lg, lb)


def _resmlp_body(x_ref, w1_ref, b1_ref, w2_ref, b2_ref, lg_ref, lb_ref, o_ref):
    x = x_ref[...]
    h = _silu(_dot(x, w1_ref[...]) + b1_ref[...])
    m = _dot(h, w2_ref[...]) + b2_ref[...]
    o_ref[...] = x + _ln(m, lg_ref[...], lb_ref[...])


def _resmlp(x, w1, b1, w2, b2, lg, lb):
    n = x.shape[0]
    return pl.pallas_call(
        _resmlp_body,
        grid=(n // BLK,),
        in_specs=[_rspec(), _wspec(), _vspec(), _wspec(), _vspec(), _vspec(),
                  _vspec()],
        out_specs=_rspec(),
        out_shape=jax.ShapeDtypeStruct((n, H), _F32),
    )(x, w1, b1, w2, b2, lg, lb)


def _latent_body(x_ref, n_ref, w1_ref, b1_ref, w2_ref, b2_ref, mean_ref,
                 samp_ref):
    h = _silu(_dot(x_ref[...], w1_ref[...]) + b1_ref[...])
    m = _dot(h, w2_ref[...]) + b2_ref[...]
    mean_ref[...] = m
    samp_ref[...] = m + n_ref[...]


def _latent(x, noise, w1, b1, w2, b2):
    n = x.shape[0]
    return pl.pallas_call(
        _latent_body,
        grid=(n // BLK,),
        in_specs=[_rspec(), _rspec(), _wspec(), _vspec(), _wspec(), _vspec()],
        out_specs=[_rspec(), _rspec()],
        out_shape=[jax.ShapeDtypeStruct((n, H), _F32)] * 2,
    )(x, noise, w1, b1, w2, b2)


def _pmap_body(x_ref, w1_ref, b1_ref, w2_ref, b2_ref, o_ref):
    h = _silu(_dot(x_ref[...], w1_ref[...]) + b1_ref[...])
    m = _dot(h, w2_ref[...]) + b2_ref[...]
    col = lax.broadcasted_iota(jnp.int32, m.shape, 1)
    sp = jnp.maximum(m, 0.0) + jnp.log(1.0 + jnp.exp(-jnp.abs(m)))
    o_ref[...] = jnp.where(col >= 17, sp, m)


def _pmap(x, w1, b1, w2, b2):
    n = x.shape[0]
    ko = w2.shape[1]
    return pl.pallas_call(
        _pmap_body,
        grid=(n // BLK,),
        in_specs=[_rspec(), _wspec(), _vspec(),
                  pl.BlockSpec((H, ko), lambda i: (0, 0)),
                  pl.BlockSpec((1, ko), lambda i: (0, 0))],
        out_specs=pl.BlockSpec((BLK, ko), lambda i: (i, 0)),
        out_shape=jax.ShapeDtypeStruct((n, ko), _F32),
    )(x, w1, b1, w2, b2)


def _sc_mesh():
    return plsc.VectorSubcoreMesh(core_axis_name="c", subcore_axis_name="s",
                                  num_cores=NC, num_subcores=NS)


def _gather2(xs, xr, src, dst):
    e_pad = src.shape[0]
    rows_pw = e_pad // NW
    n = rows_pw // CG

    @functools.partial(
        pl.kernel,
        mesh=_sc_mesh(),
        out_type=jax.ShapeDtypeStruct((e_pad, H), _F32),
        scratch_types=[
            pltpu.VMEM((CG,), jnp.int32),
            pltpu.VMEM((CG,), jnp.int32),
            pltpu.VMEM((CG, H), _F32),
            pltpu.VMEM((CG, H), _F32),
            pltpu.SemaphoreType.DMA,
            pltpu.SemaphoreType.DMA,
        ],
    )
    def k(xs_hbm, xr_hbm, src_hbm, dst_hbm, out_hbm, ia, ib, ba, bb, sa, sb):
        wid = lax.axis_index("s") * NC + lax.axis_index("c")
        base = wid * rows_pw

        def chunk(j, carry):
            off = base + j * CG
            pltpu.sync_copy(src_hbm.at[pl.ds(off, CG)], ia)
            pltpu.sync_copy(dst_hbm.at[pl.ds(off, CG)], ib)
            ca = pltpu.async_copy(xs_hbm.at[ia], ba, sa)
            cb = pltpu.async_copy(xr_hbm.at[ib], bb, sb)
            ca.wait()
            cb.wait()

            def rows2(r, c2):
                for rr in range(2):
                    row = r * 2 + rr
                    for i in range(H // 16):
                        sl = pl.ds(i * 16, 16)
                        ba[row, sl] = ba[row, sl] + bb[row, sl]
                return c2

            lax.fori_loop(0, CG // 2, rows2, 0)
            pltpu.sync_copy(ba, out_hbm.at[pl.ds(off, CG)])
            return carry

        lax.fori_loop(0, n, chunk, 0)

    return k(xs, xr, src, dst)


def _segsum2(msg, dst, n_acc):
    e_pad = dst.shape[0]
    half = e_pad // 2
    rows_pt = half // NS
    n = rows_pt // CS
    nz = n_acc // CS
    n_zc = (nz + NS - 1) // NS

    @functools.partial(
        pl.kernel,
        mesh=_sc_mesh(),
        out_type=jax.ShapeDtypeStruct((2, n_acc, H), _F32),
        scratch_types=[
            pltpu.VMEM((CS,), jnp.int32),
            pltpu.VMEM((CS, H), _F32),
            pltpu.VMEM((CS, H), _F32),
            pltpu.VMEM_SHARED((n_acc, H), _F32),
        ],
    )
    def k(msg_hbm, dst_hbm, zero_hbm, out_hbm, idx, buf, zbuf, accum):
        c = lax.axis_index("c")
        s = lax.axis_index("s")
        pltpu.sync_copy(zero_hbm, zbuf)

        def zc(j, carry):
            ch = s + j * NS

            @pl.when(ch < nz)
            def _():
                pltpu.sync_copy(zbuf, accum.at[pl.ds(ch * CS, CS)])

            return carry

        lax.fori_loop(0, n_zc, zc, 0)
        plsc.subcore_barrier()
        base = c * half + s * rows_pt

        def chunk(j, carry):
            off = base + j * CS
            pltpu.sync_copy(dst_hbm.at[pl.ds(off, CS)], idx)
            pltpu.sync_copy(msg_hbm.at[pl.ds(off, CS)], buf)
            pltpu.sync_copy(buf, accum.at[idx], add=True)
            return carry

        lax.fori_loop(0, n, chunk, 0)
        plsc.subcore_barrier()

        def cp(j, carry):
            ch = s + j * NS

            @pl.when(ch < nz)
            def _():
                pltpu.sync_copy(accum.at[pl.ds(ch * CS, CS)],
                                out_hbm.at[c, pl.ds(ch * CS, CS)])

            return carry

        lax.fori_loop(0, n_zc, cp, 0)

    zero = jnp.zeros((CS, H), _F32)
    return k(msg, dst, zero)


def _segsum_grid(msg, dst, q):
    e_pad = dst.shape[0]
    rows_pt = e_pad // NS
    n = rows_pt // CS
    n_acc = q + CS
    nz = n_acc // CS
    n_zc = (nz + NS - 1) // NS
    nq = q // CS
    n_qc = (nq + NS - 1) // NS

    @functools.partial(
        pl.kernel,
        mesh=_sc_mesh(),
        out_type=jax.ShapeDtypeStruct((4 * q, H), _F32),
        scratch_types=[
            pltpu.VMEM((CS,), jnp.int32),
            pltpu.VMEM((CS, H), _F32),
            pltpu.VMEM((CS, H), _F32),
            pltpu.VMEM_SHARED((n_acc, H), _F32),
        ],
    )
    def k(msg_hbm, dst_hbm, zero_hbm, out_hbm, idx, buf, zbuf, accum):
        c = lax.axis_index("c")
        s = lax.axis_index("s")
        pltpu.sync_copy(zero_hbm, zbuf)
        base = s * rows_pt

        def quarter(qi, carry):
            qbase = (c * 2 + qi) * q

            def zc(j, c2):
                ch = s + j * NS

                @pl.when(ch < nz)
                def _():
                    pltpu.sync_copy(zbuf, accum.at[pl.ds(ch * CS, CS)])

                return c2

            lax.fori_loop(0, n_zc, zc, 0)
            plsc.subcore_barrier()

            def chunk(j, c2):
                off = base + j * CS
                pltpu.sync_copy(dst_hbm.at[pl.ds(off, CS)], idx)
                for i in range(CS // 16):
                    sl = pl.ds(i * 16, 16)
                    v = idx[sl] - qbase
                    ok = (v >= 0) & (v < q)
                    idx[sl] = jnp.where(ok, v, q + 8)
                pltpu.sync_copy(msg_hbm.at[pl.ds(off, CS)], buf)
                pltpu.sync_copy(buf, accum.at[idx], add=True)
                return c2

            lax.fori_loop(0, n, chunk, 0)
            plsc.subcore_barrier()

            def cp(j, c2):
                ch = s + j * NS

                @pl.when(ch < nq)
                def _():
                    pltpu.sync_copy(accum.at[pl.ds(ch * CS, CS)],
                                    out_hbm.at[pl.ds(qbase + ch * CS, CS)])

                return c2

            lax.fori_loop(0, n_qc, cp, 0)
            plsc.subcore_barrier()
            return carry

        lax.fori_loop(0, 2, quarter, 0)

    zero = jnp.zeros((CS, H), _F32)
    return k(msg, dst, zero)


def _vec(b):
    return b.reshape(1, -1)


def _interact(p, send, rec, edge, src, dst_g, dst_s, n_acc,
              update_edges=False, same=False):
    w0, b0 = p["edge_mlp"]["layers"][0]
    w1, b1 = p["edge_mlp"]["layers"][1]
    lg, lb = p["edge_mlp"]["ln"]
    w0e, w0s, w0r = w0[:H], w0[H:2 * H], w0[2 * H:]
    if same:
        xs, xr = _tables(send, w0s, w0r)
    else:
        xs = _mm(send, w0s)
        xr = _mm(rec, w0r)
    g = _gather2(xs, xr, src, dst_g)
    res = _edge_mlp(edge, g, w0e, _vec(b0), w1, _vec(b1), _vec(lg), _vec(lb),
                    update_edges)
    msg, new_edge = res if update_edges else (res, None)
    parts = _segsum2(msg, dst_s, n_acc)
    nw0, nb0 = p["node_mlp"]["layers"][0]
    nw1, nb1 = p["node_mlp"]["layers"][1]
    nlg, nlb = p["node_mlp"]["ln"]
    new_rec = _node2(rec, parts, nw0[:H], nw0[H:], _vec(nb0), nw1, _vec(nb1),
                     _vec(nlg), _vec(nlb))
    if update_edges:
        return new_rec, new_edge
    return new_rec


def kernel(grid_rep, mesh_emb_0, mesh_emb_1, g2m_emb, m2m_emb_0, m2m_emb_1,
           mesh_up_emb_0, mesh_down_emb_0, m2g_emb, latent_noise, params,
           g2m_src, g2m_dst, m2m0_src, m2m0_dst, m2m1_src, m2m1_dst,
           up0_src, up0_dst, down0_src, down0_dst, m2g_src, m2g_dst):
    n_grid, n_m0, n_m1 = (grid_rep.shape[0], mesh_emb_0.shape[0],
                          mesh_emb_1.shape[0])
    q_grid = _rup(-(-n_grid // 4), 128)
    pg = 4 * q_grid
    p0 = _rup(n_m0 + 8, BLK)
    p1 = _rup(n_m1 + 8, BLK)
    ec = NW * 128

    def pe(e):
        return _rup(e.shape[0], ec)

    grid_p = _pad_rows(grid_rep, pg)
    m0_p = _pad_rows(mesh_emb_0, p0)
    m1_p = _pad_rows(mesh_emb_1, p1)
    noise_p = _pad_rows(latent_noise, p1)
    g2m_e = _pad_rows(g2m_emb, pe(g2m_emb))
    m2m0_e = _pad_rows(m2m_emb_0, pe(m2m_emb_0))
    m2m1_e = _pad_rows(m2m_emb_1, pe(m2m_emb_1))
    up_e = _pad_rows(mesh_up_emb_0, pe(mesh_up_emb_0))
    down_e = _pad_rows(mesh_down_emb_0, pe(mesh_down_emb_0))
    m2g_e = _pad_rows(m2g_emb, pe(m2g_emb))

    def pidx(src, dst, epad, dump):
        return (_pad_idx(src, epad, 0), _pad_idx(dst, epad, 0),
                _pad_idx(dst, epad, dump))

    g2m_s, g2m_dg, g2m_ds = pidx(g2m_src, g2m_dst, g2m_e.shape[0], p0 - 8)
    m2m0_s, m2m0_dg, m2m0_ds = pidx(m2m0_src, m2m0_dst, m2m0_e.shape[0],
                                    p0 - 8)
    m2m1_s, m2m1_dg, m2m1_ds = pidx(m2m1_src, m2m1_dst, m2m1_e.shape[0],
                                    p1 - 8)
    up_s, up_dg, up_ds = pidx(up0_src, up0_dst, up_e.shape[0], p1 - 8)
    down_s, down_dg, down_ds = pidx(down0_src, down0_dst, down_e.shape[0],
                                    p0 - 8)
    m2g_s, m2g_dg, m2g_ds = pidx(m2g_src, m2g_dst, m2g_e.shape[0], pg - 8)

    gp = params["grid_update_mlp"]
    (gw1, gb1), (gw2, gb2) = gp["layers"]
    glg, glb = gp["ln"]
    residual = _resmlp(grid_p, gw1, _vec(gb1), gw2, _vec(gb2), _vec(glg),
                       _vec(glb))

    cur0 = _interact(params["g2m"], grid_p, m0_p, g2m_e, g2m_s, g2m_dg,
                     g2m_ds, p0)
    m2m0 = m2m0_e
    for p in params["intra_up_0"]:
        cur0, m2m0 = _interact(p, cur0, cur0, m2m0, m2m0_s, m2m0_dg, m2m0_ds,
                               p0, update_edges=True, same=True)
    mesh_rep_0, m2m_rep_0 = cur0, m2m0

    cur1 = _interact(params["up_0"], cur0, m1_p, up_e, up_s, up_dg, up_ds, p1)
    for p in params["intra_up_1"]:
        cur1, _ = _interact(p, cur1, cur1, m2m1_e, m2m1_s, m2m1_dg, m2m1_ds,
                            p1, update_edges=True, same=True)

    lp = params["latent_param_map"]
    (lw1, lb1), (lw2, lb2) = lp["layers"]
    latent_mean_p, samples = _latent(cur1, noise_p, lw1, _vec(lb1), lw2,
                                     _vec(lb2))

    cur0 = _interact(params["down_0"], samples, mesh_rep_0, down_e, down_s,
                     down_dg, down_ds, p0)
    for p in params["intra_down_0"]:
        cur0 = _interact(p, cur0, cur0, m2m_rep_0, m2m0_s, m2m0_dg, m2m0_ds,
                         p0, same=True)

    p = params["m2g"]
    w0, b0 = p["edge_mlp"]["layers"][0]
    w1, b1 = p["edge_mlp"]["layers"][1]
    lg, lb = p["edge_mlp"]["ln"]
    xs = _mm(cur0, w0[H:2 * H])
    xr = _mm(residual, w0[2 * H:])
    g = _gather2(xs, xr, m2g_s, m2g_dg)
    msg = _edge_mlp(m2g_e, g, w0[:H], _vec(b0), w1, _vec(b1), _vec(lg),
                    _vec(lb), False)
    agg = _segsum_grid(msg, m2g_ds, q_grid)
    nw0, nb0 = p["node_mlp"]["layers"][0]
    nw1, nb1 = p["node_mlp"]["layers"][1]
    nlg, nlb = p["node_mlp"]["ln"]
    decoded = _node1(residual, agg, nw0[:H], nw0[H:], _vec(nb0), nw1,
                     _vec(nb1), _vec(nlg), _vec(nlb))

    pp = params["param_map"]
    (pw1, pb1), (pw2, pb2) = pp["layers"]
    ko = 64
    pw2p = jnp.pad(pw2, ((0, 0), (0, ko - pw2.shape[1])))
    pb2p = jnp.pad(pb2, (0, ko - pb2.shape[0]))
    sp = _pmap(decoded, pw1, _vec(pb1), pw2p, _vec(pb2p))

    latent_mean = latent_mean_p[:n_m1]
    mean = sp[:n_grid, :17]
    std = sp[:n_grid, 17:34]
    return latent_mean, mean, std


# speedup vs baseline: 1.1585x; 1.1585x over previous
import functools

import jax
import jax.numpy as jnp
from jax import lax
from jax.experimental import pallas as pl
from jax.experimental.pallas import tpu as pltpu
from jax.experimental.pallas import tpu_sc as plsc

H = 128
NC, NS = 2, 16
NW = NC * NS
CG = 128
CS = 64
BLK = 512

_F32 = jnp.float32


def _rup(x, m):
    return (x + m - 1) // m * m


def _pad_rows(x, p):
    return jnp.pad(x, ((0, p - x.shape[0]), (0, 0)))


def _pad_idx(x, p, val):
    return jnp.pad(x, (0, p - x.shape[0]), constant_values=val)


def _silu(x):
    return x * (1.0 / (1.0 + jnp.exp(-x)))


def _ln(x, g, b):
    mu = jnp.mean(x, axis=-1, keepdims=True)
    var = jnp.mean((x - mu) ** 2, axis=-1, keepdims=True)
    return (x - mu) / jnp.sqrt(var + 1e-5) * g + b


def _dot(a, b):
    return jnp.dot(a, b, preferred_element_type=_F32)


def _wspec():
    return pl.BlockSpec((H, H), lambda i: (0, 0))


def _vspec():
    return pl.BlockSpec((1, H), lambda i: (0, 0))


def _rspec(blk=BLK):
    return pl.BlockSpec((blk, H), lambda i: (i, 0))


def _mm_body(x_ref, w_ref, o_ref):
    o_ref[...] = _dot(x_ref[...], w_ref[...])


def _mm(x, w):
    n = x.shape[0]
    return pl.pallas_call(
        _mm_body,
        grid=(n // BLK,),
        in_specs=[_rspec(), _wspec()],
        out_specs=_rspec(),
        out_shape=jax.ShapeDtypeStruct((n, H), _F32),
    )(x, w)


def _tables_body(x_ref, ws_ref, wr_ref, xs_ref, xr_ref):
    x = x_ref[...]
    xs_ref[...] = _dot(x, ws_ref[...])
    xr_ref[...] = _dot(x, wr_ref[...])


def _tables(x, ws, wr):
    n = x.shape[0]
    return pl.pallas_call(
        _tables_body,
        grid=(n // BLK,),
        in_specs=[_rspec(), _wspec(), _wspec()],
        out_specs=[_rspec(), _rspec()],
        out_shape=[jax.ShapeDtypeStruct((n, H), _F32)] * 2,
    )(x, ws, wr)


def _edge_core(e_ref, g_ref, w1_ref, b1_ref, w2_ref, b2_ref, lg_ref, lb_ref):
    h = _dot(e_ref[...], w1_ref[...]) + g_ref[...] + b1_ref[...]
    h = _silu(h)
    m = _dot(h, w2_ref[...]) + b2_ref[...]
    return _ln(m, lg_ref[...], lb_ref[...])


def _edge_body(e_ref, g_ref, w1_ref, b1_ref, w2_ref, b2_ref, lg_ref, lb_ref,
               msg_ref):
    msg_ref[...] = _edge_core(e_ref, g_ref, w1_ref, b1_ref, w2_ref, b2_ref,
                              lg_ref, lb_ref)


def _edge_upd_body(e_ref, g_ref, w1_ref, b1_ref, w2_ref, b2_ref, lg_ref,
                   lb_ref, msg_ref, ne_ref):
    m = _edge_core(e_ref, g_ref, w1_ref, b1_ref, w2_ref, b2_ref, lg_ref,
                   lb_ref)
    msg_ref[...] = m
    ne_ref[...] = e_ref[...] + m


def _edge_mlp(edge, g, w1e, b1, w2, b2, lg, lb, update_edges):
    n = edge.shape[0]
    specs = [_rspec(), _rspec(), _wspec(), _vspec(), _wspec(), _vspec(),
             _vspec(), _vspec()]
    if update_edges:
        return pl.pallas_call(
            _edge_upd_body,
            grid=(n // BLK,),
            in_specs=specs,
            out_specs=[_rspec(), _rspec()],
            out_shape=[jax.ShapeDtypeStruct((n, H), _F32)] * 2,
        )(edge, g, w1e, b1, w2, b2, lg, lb)
    return pl.pallas_call(
        _edge_body,
        grid=(n // BLK,),
        in_specs=specs,
        out_specs=_rspec(),
        out_shape=jax.ShapeDtypeStruct((n, H), _F32),
    )(edge, g, w1e, b1, w2, b2, lg, lb)


def _node_core(rec, agg, wr_ref, wa_ref, b1_ref, w2_ref, b2_ref, lg_ref,
               lb_ref):
    h = _silu(_dot(rec, wr_ref[...]) + _dot(agg, wa_ref[...]) + b1_ref[...])
    m = _dot(h, w2_ref[...]) + b2_ref[...]
    return rec + _ln(m, lg_ref[...], lb_ref[...])


def _node2_body(rec_ref, parts_ref, wr_ref, wa_ref, b1_ref, w2_ref, b2_ref,
                lg_ref, lb_ref, o_ref):
    agg = parts_ref[0] + parts_ref[1]
    o_ref[...] = _node_core(rec_ref[...], agg, wr_ref, wa_ref, b1_ref, w2_ref,
                            b2_ref, lg_ref, lb_ref)


def _node2(rec, parts, wr, wa, b1, w2, b2, lg, lb):
    n = rec.shape[0]
    return pl.pallas_call(
        _node2_body,
        grid=(n // BLK,),
        in_specs=[_rspec(), pl.BlockSpec((2, BLK, H), lambda i: (0, i, 0)),
                  _wspec(), _wspec(), _vspec(), _wspec(), _vspec(), _vspec(),
                  _vspec()],
        out_specs=_rspec(),
        out_shape=jax.ShapeDtypeStruct((n, H), _F32),
    )(rec, parts, wr, wa, b1, w2, b2, lg, lb)


def _node1_body(rec_ref, agg_ref, wr_ref, wa_ref, b1_ref, w2_ref, b2_ref,
                lg_ref, lb_ref, o_ref):
    o_ref[...] = _node_core(rec_ref[...], agg_ref[...], wr_ref, wa_ref,
                            b1_ref, w2_ref, b2_ref, lg_ref, lb_ref)


def _node1(rec, agg, wr, wa, b1, w2, b2, lg, lb):
    n = rec.shape[0]
    return pl.pallas_call(
        _node1_body,
        grid=(n // BLK,),
        in_specs=[_rspec(), _rspec(), _wspec(), _wspec(), _vspec(), _wspec(),
                  _vspec(), _vspec(), _vspec()],
        out_specs=_rspec(),
        out_shape=jax.ShapeDtypeStruct((n, H), _F32),
    )(rec, agg, wr, wa, b1, w2, b2, lg, lb)


def _resmlp_body(x_ref, w1_ref, b1_ref, w2_ref, b2_ref, lg_ref, lb_ref, o_ref):
    x = x_ref[...]
    h = _silu(_dot(x, w1_ref[...]) + b1_ref[...])
    m = _dot(h, w2_ref[...]) + b2_ref[...]
    o_ref[...] = x + _ln(m, lg_ref[...], lb_ref[...])


def _resmlp(x, w1, b1, w2, b2, lg, lb):
    n = x.shape[0]
    return pl.pallas_call(
        _resmlp_body,
        grid=(n // BLK,),
        in_specs=[_rspec(), _wspec(), _vspec(), _wspec(), _vspec(), _vspec(),
                  _vspec()],
        out_specs=_rspec(),
        out_shape=jax.ShapeDtypeStruct((n, H), _F32),
    )(x, w1, b1, w2, b2, lg, lb)


def _latent_body(x_ref, n_ref, w1_ref, b1_ref, w2_ref, b2_ref, mean_ref,
                 samp_ref):
    h = _silu(_dot(x_ref[...], w1_ref[...]) + b1_ref[...])
    m = _dot(h, w2_ref[...]) + b2_ref[...]
    mean_ref[...] = m
    samp_ref[...] = m + n_ref[...]


def _latent(x, noise, w1, b1, w2, b2):
    n = x.shape[0]
    return pl.pallas_call(
        _latent_body,
        grid=(n // BLK,),
        in_specs=[_rspec(), _rspec(), _wspec(), _vspec(), _wspec(), _vspec()],
        out_specs=[_rspec(), _rspec()],
        out_shape=[jax.ShapeDtypeStruct((n, H), _F32)] * 2,
    )(x, noise, w1, b1, w2, b2)


def _pmap_body(x_ref, w1_ref, b1_ref, w2_ref, b2_ref, o_ref):
    h = _silu(_dot(x_ref[...], w1_ref[...]) + b1_ref[...])
    m = _dot(h, w2_ref[...]) + b2_ref[...]
    col = lax.broadcasted_iota(jnp.int32, m.shape, 1)
    sp = jnp.maximum(m, 0.0) + jnp.log(1.0 + jnp.exp(-jnp.abs(m)))
    o_ref[...] = jnp.where(col >= 17, sp, m)


def _pmap(x, w1, b1, w2, b2):
    n = x.shape[0]
    ko = w2.shape[1]
    return pl.pallas_call(
        _pmap_body,
        grid=(n // BLK,),
        in_specs=[_rspec(), _wspec(), _vspec(),
                  pl.BlockSpec((H, ko), lambda i: (0, 0)),
                  pl.BlockSpec((1, ko), lambda i: (0, 0))],
        out_specs=pl.BlockSpec((BLK, ko), lambda i: (i, 0)),
        out_shape=jax.ShapeDtypeStruct((n, ko), _F32),
    )(x, w1, b1, w2, b2)


def _sc_mesh():
    return plsc.VectorSubcoreMesh(core_axis_name="c", subcore_axis_name="s",
                                  num_cores=NC, num_subcores=NS)


def _gather2(xs, xr, src, dst):
    e_pad = src.shape[0]
    rows_pw = e_pad // NW
    n = rows_pw // CG

    @functools.partial(
        pl.kernel,
        mesh=_sc_mesh(),
        out_type=jax.ShapeDtypeStruct((e_pad, H), _F32),
        scratch_types=[
            pltpu.VMEM((CG,), jnp.int32),
            pltpu.VMEM((CG,), jnp.int32),
            pltpu.VMEM((CG, H), _F32),
            pltpu.VMEM((CG, H), _F32),
            pltpu.SemaphoreType.DMA,
            pltpu.SemaphoreType.DMA,
        ],
    )
    def k(xs_hbm, xr_hbm, src_hbm, dst_hbm, out_hbm, ia, ib, ba, bb, sa, sb):
        wid = lax.axis_index("s") * NC + lax.axis_index("c")
        base = wid * rows_pw

        def chunk(j, carry):
            off = base + j * CG
            pltpu.sync_copy(src_hbm.at[pl.ds(off, CG)], ia)
            pltpu.sync_copy(dst_hbm.at[pl.ds(off, CG)], ib)
            ca = pltpu.async_copy(xs_hbm.at[ia], ba, sa)
            cb = pltpu.async_copy(xr_hbm.at[ib], bb, sb)
            ca.wait()
            cb.wait()

            def rows2(r, c2):
                for rr in range(2):
                    row = r * 2 + rr
                    for i in range(H // 16):
                        sl = pl.ds(i * 16, 16)
                        ba[row, sl] = ba[row, sl] + bb[row, sl]
                return c2

            lax.fori_loop(0, CG // 2, rows2, 0)
            pltpu.sync_copy(ba, out_hbm.at[pl.ds(off, CG)])
            return carry

        lax.fori_loop(0, n, chunk, 0)

    return k(xs, xr, src, dst)


def _segsum2(msg, dst, n_acc):
    e_pad = dst.shape[0]
    half = e_pad // 2
    rows_pt = half // NS
    n = rows_pt // CS
    nz = n_acc // CS
    n_zc = (nz + NS - 1) // NS

    @functools.partial(
        pl.kernel,
        mesh=_sc_mesh(),
        out_type=jax.ShapeDtypeStruct((2, n_acc, H), _F32),
        scratch_types=[
            pltpu.VMEM((CS,), jnp.int32),
            pltpu.VMEM((CS, H), _F32),
            pltpu.VMEM((CS, H), _F32),
            pltpu.VMEM_SHARED((n_acc, H), _F32),
        ],
    )
    def k(msg_hbm, dst_hbm, zero_hbm, out_hbm, idx, buf, zbuf, accum):
        c = lax.axis_index("c")
        s = lax.axis_index("s")
        pltpu.sync_copy(zero_hbm, zbuf)

        def zc(j, carry):
            ch = s + j * NS

            @pl.when(ch < nz)
            def _():
                pltpu.sync_copy(zbuf, accum.at[pl.ds(ch * CS, CS)])

            return carry

        lax.fori_loop(0, n_zc, zc, 0)
        plsc.subcore_barrier()
        base = c * half + s * rows_pt

        def chunk(j, carry):
            off = base + j * CS
            pltpu.sync_copy(dst_hbm.at[pl.ds(off, CS)], idx)
            pltpu.sync_copy(msg_hbm.at[pl.ds(off, CS)], buf)
            pltpu.sync_copy(buf, accum.at[idx], add=True)
            return carry

        lax.fori_loop(0, n, chunk, 0)
        plsc.subcore_barrier()

        def cp(j, carry):
            ch = s + j * NS

            @pl.when(ch < nz)
            def _():
                pltpu.sync_copy(accum.at[pl.ds(ch * CS, CS)],
                                out_hbm.at[c, pl.ds(ch * CS, CS)])

            return carry

        lax.fori_loop(0, n_zc, cp, 0)

    zero = jnp.zeros((CS, H), _F32)
    return k(msg, dst, zero)


def _segsum_grid(msg, dst, q):
    e_pad = dst.shape[0]
    rows_pt = e_pad // NS
    n = rows_pt // CS
    n_acc = q + CS
    nz = n_acc // CS
    n_zc = (nz + NS - 1) // NS
    nq = q // CS
    n_qc = (nq + NS - 1) // NS

    @functools.partial(
        pl.kernel,
        mesh=_sc_mesh(),
        out_type=jax.ShapeDtypeStruct((4 * q, H), _F32),
        scratch_types=[
            pltpu.VMEM((CS,), jnp.int32),
            pltpu.VMEM((CS, H), _F32),
            pltpu.VMEM((CS, H), _F32),
            pltpu.VMEM_SHARED((n_acc, H), _F32),
        ],
    )
    def k(msg_hbm, dst_hbm, zero_hbm, out_hbm, idx, buf, zbuf, accum):
        c = lax.axis_index("c")
        s = lax.axis_index("s")
        pltpu.sync_copy(zero_hbm, zbuf)
        base = s * rows_pt

        def quarter(qi, carry):
            qbase = (c * 2 + qi) * q

            def zc(j, c2):
                ch = s + j * NS

                @pl.when(ch < nz)
                def _():
                    pltpu.sync_copy(zbuf, accum.at[pl.ds(ch * CS, CS)])

                return c2

            lax.fori_loop(0, n_zc, zc, 0)
            plsc.subcore_barrier()

            def chunk(j, c2):
                off = base + j * CS
                pltpu.sync_copy(dst_hbm.at[pl.ds(off, CS)], idx)
                for i in range(CS // 16):
                    sl = pl.ds(i * 16, 16)
                    v = idx[sl] - qbase
                    ok = (v >= 0) & (v < q)
                    idx[sl] = jnp.where(ok, v, q + 8)
                pltpu.sync_copy(msg_hbm.at[pl.ds(off, CS)], buf)
                pltpu.sync_copy(buf, accum.at[idx], add=True)
                return c2

            lax.fori_loop(0, n, chunk, 0)
            plsc.subcore_barrier()

            def cp(j, c2):
                ch = s + j * NS

                @pl.when(ch < nq)
                def _():
                    pltpu.sync_copy(accum.at[pl.ds(ch * CS, CS)],
                                    out_hbm.at[pl.ds(qbase + ch * CS, CS)])

                return c2

            lax.fori_loop(0, n_qc, cp, 0)
            plsc.subcore_barrier()
            return carry

        lax.fori_loop(0, 2, quarter, 0)

    zero = jnp.zeros((CS, H), _F32)
    return k(msg, dst, zero)


def _vec(b):
    return b.reshape(1, -1)


def _interact(p, send, rec, edge, src, dst_g, dst_s, n_acc,
              update_edges=False, same=False):
    w0, b0 = p["edge_mlp"]["layers"][0]
    w1, b1 = p["edge_mlp"]["layers"][1]
    lg, lb = p["edge_mlp"]["ln"]
    w0e, w0s, w0r = w0[:H], w0[H:2 * H], w0[2 * H:]
    if same:
        xs, xr = _tables(send, w0s, w0r)
    else:
        xs = _mm(send, w0s)
        xr = _mm(rec, w0r)
    g = _gather2(xs, xr, src, dst_g)
    res = _edge_mlp(edge, g, w0e, _vec(b0), w1, _vec(b1), _vec(lg), _vec(lb),
                    update_edges)
    msg, new_edge = res if update_edges else (res, None)
    parts = _segsum2(msg, dst_s, n_acc)
    nw0, nb0 = p["node_mlp"]["layers"][0]
    nw1, nb1 = p["node_mlp"]["layers"][1]
    nlg, nlb = p["node_mlp"]["ln"]
    new_rec = _node2(rec, parts, nw0[:H], nw0[H:], _vec(nb0), nw1, _vec(nb1),
                     _vec(nlg), _vec(nlb))
    if update_edges:
        return new_rec, new_edge
    return new_rec


def kernel(grid_rep, mesh_emb_0, mesh_emb_1, g2m_emb, m2m_emb_0, m2m_emb_1,
           mesh_up_emb_0, mesh_down_emb_0, m2g_emb, latent_noise, params,
           g2m_src, g2m_dst, m2m0_src, m2m0_dst, m2m1_src, m2m1_dst,
           up0_src, up0_dst, down0_src, down0_dst, m2g_src, m2g_dst):
    n_grid, n_m0, n_m1 = (grid_rep.shape[0], mesh_emb_0.shape[0],
                          mesh_emb_1.shape[0])
    q_grid = _rup(-(-n_grid // 4), 128)
    pg = 4 * q_grid
    p0 = _rup(n_m0 + 8, BLK)
    p1 = _rup(n_m1 + 8, BLK)
    ec = NW * 128

    def pe(e):
        return _rup(e.shape[0], ec)

    grid_p = _pad_rows(grid_rep, pg)
    m0_p = _pad_rows(mesh_emb_0, p0)
    m1_p = _pad_rows(mesh_emb_1, p1)
    noise_p = _pad_rows(latent_noise, p1)
    g2m_e = _pad_rows(g2m_emb, pe(g2m_emb))
    m2m0_e = _pad_rows(m2m_emb_0, pe(m2m_emb_0))
    m2m1_e = _pad_rows(m2m_emb_1, pe(m2m_emb_1))
    up_e = _pad_rows(mesh_up_emb_0, pe(mesh_up_emb_0))
    down_e = _pad_rows(mesh_down_emb_0, pe(mesh_down_emb_0))
    m2g_e = _pad_rows(m2g_emb, pe(m2g_emb))

    def pidx(src, dst, epad, dump):
        return (_pad_idx(src, epad, 0), _pad_idx(dst, epad, 0),
                _pad_idx(dst, epad, dump))

    g2m_s, g2m_dg, g2m_ds = pidx(g2m_src, g2m_dst, g2m_e.shape[0], p0 - 8)
    m2m0_s, m2m0_dg, m2m0_ds = pidx(m2m0_src, m2m0_dst, m2m0_e.shape[0],
                                    p0 - 8)
    m2m1_s, m2m1_dg, m2m1_ds = pidx(m2m1_src, m2m1_dst, m2m1_e.shape[0],
                                    p1 - 8)
    up_s, up_dg, up_ds = pidx(up0_src, up0_dst, up_e.shape[0], p1 - 8)
    down_s, down_dg, down_ds = pidx(down0_src, down0_dst, down_e.shape[0],
                                    p0 - 8)
    m2g_s, m2g_dg, m2g_ds = pidx(m2g_src, m2g_dst, m2g_e.shape[0], pg - 8)

    gp = params["grid_update_mlp"]
    (gw1, gb1), (gw2, gb2) = gp["layers"]
    glg, glb = gp["ln"]
    residual = _resmlp(grid_p, gw1, _vec(gb1), gw2, _vec(gb2), _vec(glg),
                       _vec(glb))

    cur0 = _interact(params["g2m"], grid_p, m0_p, g2m_e, g2m_s, g2m_dg,
                     g2m_ds, p0)
    m2m0 = m2m0_e
    for p in params["intra_up_0"]:
        cur0, m2m0 = _interact(p, cur0, cur0, m2m0, m2m0_s, m2m0_dg, m2m0_ds,
                               p0, update_edges=True, same=True)
    mesh_rep_0, m2m_rep_0 = cur0, m2m0

    cur1 = _interact(params["up_0"], cur0, m1_p, up_e, up_s, up_dg, up_ds, p1)
    for p in params["intra_up_1"]:
        cur1, _ = _interact(p, cur1, cur1, m2m1_e, m2m1_s, m2m1_dg, m2m1_ds,
                            p1, update_edges=True, same=True)

    lp = params["latent_param_map"]
    (lw1, lb1), (lw2, lb2) = lp["layers"]
    latent_mean_p, samples = _latent(cur1, noise_p, lw1, _vec(lb1), lw2,
                                     _vec(lb2))

    cur0 = _interact(params["down_0"], samples, mesh_rep_0, down_e, down_s,
                     down_dg, down_ds, p0)
    for p in params["intra_down_0"]:
        cur0 = _interact(p, cur0, cur0, m2m_rep_0, m2m0_s, m2m0_dg, m2m0_ds,
                         p0, same=True)

    p = params["m2g"]
    w0, b0 = p["edge_mlp"]["layers"][0]
    w1, b1 = p["edge_mlp"]["layers"][1]
    lg, lb = p["edge_mlp"]["ln"]
    xs = _mm(cur0, w0[H:2 * H])
    xr = _mm(residual, w0[2 * H:])
    g = _gather2(xs, xr, m2g_s, m2g_dg)
    msg = _edge_mlp(m2g_e, g, w0[:H], _vec(b0), w1, _vec(b1), _vec(lg),
                    _vec(lb), False)
    agg = _segsum_grid(msg, m2g_ds, q_grid)
    nw0, nb0 = p["node_mlp"]["layers"][0]
    nw1, nb1 = p["node_mlp"]["layers"][1]
    nlg, nlb = p["node_mlp"]["ln"]
    decoded = _node1(residual, agg, nw0[:H], nw0[H:], _vec(nb0), nw1,
                     _vec(nb1), _vec(nlg), _vec(nlb))

    pp = params["param_map"]
    (pw1, pb1), (pw2, pb2) = pp["layers"]
    ko = 64
    pw2p = jnp.pad(pw2, ((0, 0), (0, ko - pw2.shape[1])))
    pb2p = jnp.pad(pb2, (0, ko - pb2.shape[0]))
    sp = _pmap(decoded, pw1, _vec(pb1), pw2p, _vec(pb2p))

    latent_mean = latent_mean_p[:n_m1]
    mean = sp[:n_grid, :17]
    std = sp[:n_grid, 17:34]
    return latent_mean, mean, std
